# baseline (speedup 1.0000x reference)
import functools

import jax
import jax.numpy as jnp
from jax import lax
from jax.experimental import pallas as pl
from jax.experimental.pallas import tpu as pltpu

try:
    from jax.experimental.pallas import tpu_sc as plsc
except ImportError:
    plsc = None

_K = 64
_NROWS = 512
_M = 32768
_ROWS = 64
_NW = 32
_RPW = _NROWS // _NW
_VECS = _M // 16


def _sc_thresholds(x2d):
    mesh = plsc.VectorSubcoreMesh(core_axis_name="c", subcore_axis_name="s")

    @functools.partial(
        pl.kernel,
        mesh=mesh,
        out_type=jax.ShapeDtypeStruct((_NROWS,), jnp.float32),
        scratch_types=[
            pltpu.VMEM((_M,), jnp.float32),
            pltpu.VMEM((16 * 256,), jnp.int32),
            pltpu.VMEM((_RPW,), jnp.float32),
        ],
        compiler_params=pltpu.CompilerParams(needs_layout_passes=False),
    )
    def k(x_hbm, out_hbm, row_v, hist, outbuf):
        wid = lax.axis_index("s") * 2 + lax.axis_index("c")
        lanes = lax.iota(jnp.int32, 16)
        laneoff = lanes * jnp.int32(256)
        ones16 = jnp.ones((16,), jnp.int32)
        zeros16 = jnp.zeros((16,), jnp.int32)
        kvec = jnp.full((16,), jnp.int32(_K))

        def do_row(r, _):
            row = wid * _RPW + r
            pltpu.sync_copy(x_hbm.at[row], row_v)

            prefix = zeros16
            rk = kvec

            for rnd, sh in enumerate((24, 16, 8, 0)):
                for g2 in range(256):
                    hist[pl.ds(g2 * 16, 16)] = zeros16

                def scan_body(i, carry, _sh=sh, _rnd=rnd):
                    pfx, _ = carry
                    v = row_v[pl.ds(i * 16, 16)]
                    iv = lax.bitcast_convert_type(v, jnp.int32)
                    ub = iv ^ ((iv >> 31) | jnp.int32(-2147483648))
                    digit = lax.shift_right_logical(ub, _sh) & jnp.int32(255)
                    if _rnd == 0:
                        plsc.addupdate_scatter(hist, [laneoff + digit], ones16)
                    else:
                        act = lax.shift_right_logical(ub, _sh + 8) == pfx
                        plsc.addupdate_scatter(hist, [laneoff + digit],
                                               ones16, mask=act)
                    return carry

                lax.fori_loop(0, _VECS, scan_body, (prefix, 0))

                run = zeros16
                found = zeros16 > jnp.int32(0)
                dsel = zeros16
                cnt_gt = zeros16
                for g in range(15, -1, -1):
                    mg = hist[pl.ds(g * 16, 16)]
                    for l in range(1, 16):
                        mg = mg + hist[pl.ds(l * 256 + g * 16, 16)]
                    sfx = lax.rev(plsc.cumsum(lax.rev(mg, (0,))), (0,)) + run
                    ok = sfx >= rk
                    npc = plsc.all_reduce_population_count(ok)
                    hit = jnp.logical_not(found) & (npc > 0)
                    dl = npc - 1
                    sfx_d = jnp.max(jnp.where(lanes == dl, sfx,
                                              jnp.int32(-2147483648)))
                    mg_d = jnp.max(jnp.where(lanes == dl, mg,
                                             jnp.int32(-2147483648)))
                    dsel = jnp.where(hit, g * 16 + dl, dsel)
                    cnt_gt = jnp.where(hit, sfx_d - mg_d, cnt_gt)
                    found = found | (npc > 0)
                    run = jnp.max(jnp.where(lanes == 0, sfx,
                                            jnp.int32(-2147483648)))

                prefix = (prefix << 8) | dsel
                rk = rk - cnt_gt

            fb = prefix ^ (((~prefix) >> 31) | jnp.int32(-2147483648))
            vf = lax.bitcast_convert_type(fb, jnp.float32)
            plsc.store_scatter(outbuf, [jnp.full((16,), r, jnp.int32)], vf,
                               mask=lanes == 0)
            return _

        lax.fori_loop(0, _RPW, do_row, 0)
        pltpu.sync_copy(outbuf, out_hbm.at[pl.ds(wid * _RPW, _RPW)])

    return k(x2d)


def _tc_apply_body(x_ref, t_ref, o_ref):
    x = x_ref[0]
    pf = t_ref[0]
    kf = jnp.float32(_K)

    gt = x > pf
    eq = x == pf
    cnt_gt = jnp.sum(gt.astype(jnp.float32), axis=-1, keepdims=True)
    cnt_eq = jnp.sum(eq.astype(jnp.float32), axis=-1, keepdims=True)
    r = kf - cnt_gt
    exact = cnt_gt + cnt_eq == kf
    idx = lax.broadcasted_iota(jnp.int32, (_ROWS, _M), 1)
    all_exact = jnp.all(exact)

    def cond2(carry):
        b2, _ = carry
        return (b2 >= 0) & jnp.logical_not(all_exact)

    def body2(carry):
        b2, p2 = carry
        t2 = p2 | (jnp.int32(1) << b2)
        f = jnp.sum((eq & (idx < t2)).astype(jnp.float32), axis=-1,
                    keepdims=True)
        return b2 - 1, jnp.where(f < r, t2, p2)

    _, p2 = lax.while_loop(cond2, body2,
                           (jnp.int32(14), jnp.zeros((_ROWS, 1), jnp.int32)))
    j = jnp.where(exact, jnp.int32(_M - 1), p2)
    mask = gt | (eq & (idx <= j))

    kept = jnp.where(mask, x, jnp.float32(0.0))
    s = jnp.sum(kept, axis=-1, keepdims=True) + jnp.float32(1e-12)
    o_ref[0] = kept * (jnp.float32(1.0) / s)


def kernel(logits):
    C, L, M = logits.shape
    x2d = logits.reshape(_NROWS, M)
    thr = _sc_thresholds(x2d)

    grid = _NROWS // _ROWS
    x = logits.reshape(grid, _ROWS, M)
    t = thr.reshape(grid, _ROWS, 1)
    out = pl.pallas_call(
        _tc_apply_body,
        grid=(grid,),
        in_specs=[
            pl.BlockSpec((1, _ROWS, M), lambda g: (g, 0, 0)),
            pl.BlockSpec((1, _ROWS, 1), lambda g: (g, 0, 0)),
        ],
        out_specs=pl.BlockSpec((1, _ROWS, M), lambda g: (g, 0, 0)),
        out_shape=jax.ShapeDtypeStruct((grid, _ROWS, M), jnp.float32),
    )(x, t)
    return out.reshape(C, L, M)


# speedup vs baseline: 6.8345x; 6.8345x over previous
import jax
import jax.numpy as jnp
from jax import lax
from jax.experimental import pallas as pl

_K = 64
_ROWS = 64
_M = 32768
_CHUNKS = 64


def _key(f):
    i = lax.bitcast_convert_type(f, jnp.int32)
    return i ^ ((i >> 31) & jnp.int32(0x7FFFFFFF))


def _unkey(k):
    return lax.bitcast_convert_type(k ^ ((k >> 31) & jnp.int32(0x7FFFFFFF)),
                                    jnp.float32)


def _topk_mask_body(x_ref, o_ref):
    x = x_ref[0]
    kf = jnp.float32(_K)
    ones_col = jnp.ones((_M, 1), jnp.float32)

    def _count(mask):
        return jax.lax.dot_general(
            mask.astype(jnp.float32), ones_col,
            (((1,), (0,)), ((), ())),
            preferred_element_type=jnp.float32)

    cmax = jnp.max(x.reshape(_ROWS, _CHUNKS, _M // _CHUNKS), axis=-1)
    lo0f = jnp.min(cmax, axis=-1, keepdims=True)
    lo0 = _key(lo0f)
    hi0 = _key(jnp.max(cmax, axis=-1, keepdims=True)) + jnp.int32(1)
    cnt0 = _count(x >= lo0f)

    def cond(carry):
        it, lo, hi, cnt = carry
        return (it < 34) & jnp.logical_not(
            jnp.all((cnt == kf) | (hi - lo == 1)))

    def body(carry):
        it, lo, hi, cnt = carry
        mid = (lo >> 1) + (hi >> 1) + (lo & hi & 1)
        c = _count(x >= _unkey(mid))
        take = c >= kf
        lo = jnp.where(take, mid, lo)
        cnt = jnp.where(take, c, cnt)
        hi = jnp.where(take, hi, mid)
        return it + 1, lo, hi, cnt

    _, p, _, cnt = lax.while_loop(cond, body, (jnp.int32(0), lo0, hi0, cnt0))
    pf = _unkey(p)

    all_resolved = jnp.all(cnt == kf)
    eq = x == pf
    cnt_eq = _count(eq)
    r = kf - (cnt - cnt_eq)
    idx = lax.broadcasted_iota(jnp.int32, (_ROWS, _M), 1)

    def cond2(carry):
        b2, _ = carry
        return (b2 >= 0) & jnp.logical_not(all_resolved)

    def body2(carry):
        b2, p2 = carry
        t2 = p2 | (jnp.int32(1) << b2)
        f = _count(eq & (idx < t2))
        return b2 - 1, jnp.where(f < r, t2, p2)

    _, p2 = lax.while_loop(cond2, body2,
                           (jnp.int32(14), jnp.zeros((_ROWS, 1), jnp.int32)))
    j = jnp.where(cnt == kf, jnp.int32(_M - 1), p2)
    mask = (x > pf) | (eq & (idx <= j))

    kept = jnp.where(mask, x, jnp.float32(0.0))
    s = jnp.sum(kept, axis=-1, keepdims=True) + jnp.float32(1e-12)
    o_ref[0] = kept * (jnp.float32(1.0) / s)


def kernel(logits):
    C, L, M = logits.shape
    grid = (C * L) // _ROWS
    x = logits.reshape(grid, _ROWS, M)
    out = pl.pallas_call(
        _topk_mask_body,
        grid=(grid,),
        in_specs=[pl.BlockSpec((1, _ROWS, M), lambda g: (g, 0, 0))],
        out_specs=pl.BlockSpec((1, _ROWS, M), lambda g: (g, 0, 0)),
        out_shape=jax.ShapeDtypeStruct((grid, _ROWS, M), jnp.float32),
    )(x)
    return out.reshape(C, L, M)


# speedup vs baseline: 8.7333x; 1.2778x over previous
import jax
import jax.numpy as jnp
from jax import lax
from jax.experimental import pallas as pl

_K = 64
_ROWS = 64
_M = 32768
_CHUNKS = 64


def _key(f):
    i = lax.bitcast_convert_type(f, jnp.int32)
    return i ^ ((i >> 31) & jnp.int32(0x7FFFFFFF))


def _unkey(k):
    return lax.bitcast_convert_type(k ^ ((k >> 31) & jnp.int32(0x7FFFFFFF)),
                                    jnp.float32)


def _topk_mask_body(x_ref, o_ref):
    x = x_ref[0]
    kf = jnp.float32(_K)

    cmax = jnp.max(x.reshape(_ROWS, _CHUNKS, _M // _CHUNKS), axis=-1)
    lo0f = jnp.min(cmax, axis=-1, keepdims=True)
    lo0 = _key(lo0f)
    hi0 = _key(jnp.max(cmax, axis=-1, keepdims=True)) + jnp.int32(1)
    cnt0 = jnp.sum((x >= lo0f).astype(jnp.float32), axis=-1, keepdims=True)

    def cond(carry):
        it, lo, hi, cnt = carry
        return (it < 34) & jnp.logical_not(
            jnp.all((cnt == kf) | (hi - lo == 1)))

    def body(carry):
        it, lo, hi, cnt = carry
        mid = (lo >> 1) + (hi >> 1) + (lo & hi & 1)
        c = jnp.sum((x >= _unkey(mid)).astype(jnp.float32), axis=-1,
                    keepdims=True)
        take = c >= kf
        lo = jnp.where(take, mid, lo)
        cnt = jnp.where(take, c, cnt)
        hi = jnp.where(take, hi, mid)
        return it + 1, lo, hi, cnt

    _, p, _, cnt = lax.while_loop(cond, body, (jnp.int32(0), lo0, hi0, cnt0))
    pf = _unkey(p)

    all_resolved = jnp.all(cnt == kf)
    eq = x == pf
    idx = lax.broadcasted_iota(jnp.int32, (_ROWS, _M), 1)

    def cond2(carry):
        b2, _ = carry
        return (b2 >= 0) & jnp.logical_not(all_resolved)

    def body2(carry):
        b2, p2 = carry
        cnt_eq = jnp.sum(eq.astype(jnp.float32), axis=-1, keepdims=True)
        r = kf - (cnt - cnt_eq)
        t2 = p2 | (jnp.int32(1) << b2)
        f = jnp.sum((eq & (idx < t2)).astype(jnp.float32), axis=-1,
                    keepdims=True)
        return b2 - 1, jnp.where(f < r, t2, p2)

    _, p2 = lax.while_loop(cond2, body2,
                           (jnp.int32(14), jnp.zeros((_ROWS, 1), jnp.int32)))
    j = jnp.where(cnt == kf, jnp.int32(_M - 1), p2)
    mask = (x > pf) | (eq & (idx <= j))

    kept = jnp.where(mask, x, jnp.float32(0.0))
    s = jnp.sum(kept, axis=-1, keepdims=True) + jnp.float32(1e-12)
    o_ref[0] = kept * (jnp.float32(1.0) / s)


def kernel(logits):
    C, L, M = logits.shape
    grid = (C * L) // _ROWS
    x = logits.reshape(grid, _ROWS, M)
    out = pl.pallas_call(
        _topk_mask_body,
        grid=(grid,),
        in_specs=[pl.BlockSpec((1, _ROWS, M), lambda g: (g, 0, 0))],
        out_specs=pl.BlockSpec((1, _ROWS, M), lambda g: (g, 0, 0)),
        out_shape=jax.ShapeDtypeStruct((grid, _ROWS, M), jnp.float32),
    )(x)
    return out.reshape(C, L, M)
